# baseline (speedup 1.0000x reference)
import functools

import jax
import jax.numpy as jnp
from jax import lax
from jax.experimental import pallas as pl
from jax.experimental.pallas import tpu as pltpu
from jax.experimental.pallas import tpu_sc as plsc

_K = 128
_L = 16
_CHUNKS = _K // _L


def _sc_body(t_hbm, iv_hbm, out_hbm, t_v, iv_v, out_v):
    cid = lax.axis_index("c")
    sid = lax.axis_index("s")

    @pl.when(jnp.logical_and(cid == 0, sid == 0))
    def _():
        pltpu.sync_copy(t_hbm, t_v.at[pl.ds(0, 1)])
        pltpu.sync_copy(iv_hbm, iv_v)

        zero = jnp.zeros((_L,), jnp.int32)
        one = jnp.ones((_L,), jnp.int32)
        tv = plsc.load_gather(t_v, [zero])
        lane = lax.broadcasted_iota(jnp.int32, (_L,), 0)
        best = jnp.full((_L,), _K, dtype=jnp.int32)
        for i in range(_CHUNKS):
            row = lane + (i * _L)
            lov = plsc.load_gather(iv_v, [row, zero])
            hiv = plsc.load_gather(iv_v, [row, one])
            cond = jnp.logical_and(tv >= lov, tv <= hiv)
            ffs = plsc.all_reduce_ffs(cond)
            cand = jnp.where(ffs >= _L, _K, ffs + (i * _L))
            best = jnp.minimum(best, cand)

        res = jnp.where(best >= _K, jnp.full((_L,), -1, jnp.int32), best)
        out_v[...] = res
        pltpu.sync_copy(out_v.at[pl.ds(0, 1)], out_hbm)


@jax.jit
def _time_to_interval(tv, intervals):
    run = functools.partial(
        pl.kernel,
        out_type=jax.ShapeDtypeStruct((1,), jnp.int32),
        mesh=plsc.VectorSubcoreMesh(
            core_axis_name="c", subcore_axis_name="s", num_cores=1,
            num_subcores=1
        ),
        compiler_params=pltpu.CompilerParams(
            needs_layout_passes=False, skip_device_barrier=True
        ),
        scratch_types=[
            pltpu.VMEM((_L,), jnp.float32),
            pltpu.VMEM((_K, 2), jnp.float32),
            pltpu.VMEM((_L,), jnp.int32),
        ],
    )(_sc_body)
    return run(tv, intervals)


def kernel(t, intervals):
    tv = jnp.reshape(jnp.asarray(t, jnp.float32), (1,))
    out = _time_to_interval(tv, jnp.asarray(intervals, jnp.float32))
    return jnp.reshape(out, ())


# speedup vs baseline: 1.0708x; 1.0708x over previous
import functools

import jax
import jax.numpy as jnp
from jax import lax
from jax.experimental import pallas as pl
from jax.experimental.pallas import tpu as pltpu
from jax.experimental.pallas import tpu_sc as plsc

_K = 128
_L = 16
_CHUNKS = _K // _L


def _sc_body(t_hbm, flat_hbm, out_hbm, t_v, flat_v, out_v, sem_t, sem_iv):
    cid = lax.axis_index("c")
    sid = lax.axis_index("s")

    @pl.when(jnp.logical_and(cid == 0, sid == 0))
    def _():
        cp_t = pltpu.make_async_copy(t_hbm, t_v.at[pl.ds(0, 1)], sem_t)
        cp_iv = pltpu.make_async_copy(flat_hbm, flat_v, sem_iv)
        cp_t.start()
        cp_iv.start()
        cp_t.wait()
        cp_iv.wait()

        zero = jnp.zeros((_L,), jnp.int32)
        tv = plsc.load_gather(t_v, [zero])
        even = 2 * lax.broadcasted_iota(jnp.int32, (_L,), 0)
        best = jnp.full((_L,), _K, dtype=jnp.int32)
        for i in range(_CHUNKS):
            lov = plsc.load_gather(flat_v, [even + 2 * _L * i])
            hiv = plsc.load_gather(flat_v, [even + (2 * _L * i + 1)])
            cond = jnp.logical_and(tv >= lov, tv <= hiv)
            ffs = plsc.all_reduce_ffs(cond)
            cand = jnp.where(ffs >= _L, _K, ffs + (i * _L))
            best = jnp.minimum(best, cand)

        res = jnp.where(best >= _K, jnp.full((_L,), -1, jnp.int32), best)
        out_v[...] = res
        pltpu.sync_copy(out_v.at[pl.ds(0, 1)], out_hbm)


@jax.jit
def _time_to_interval(tv, flat):
    run = functools.partial(
        pl.kernel,
        out_type=jax.ShapeDtypeStruct((1,), jnp.int32),
        mesh=plsc.VectorSubcoreMesh(
            core_axis_name="c", subcore_axis_name="s", num_cores=1,
            num_subcores=1
        ),
        compiler_params=pltpu.CompilerParams(
            needs_layout_passes=False, skip_device_barrier=True
        ),
        scratch_types=[
            pltpu.VMEM((_L,), jnp.float32),
            pltpu.VMEM((2 * _K,), jnp.float32),
            pltpu.VMEM((_L,), jnp.int32),
            pltpu.SemaphoreType.DMA,
            pltpu.SemaphoreType.DMA,
        ],
    )(_sc_body)
    return run(tv, flat)


def kernel(t, intervals):
    tv = jnp.reshape(jnp.asarray(t, jnp.float32), (1,))
    flat = jnp.reshape(jnp.asarray(intervals, jnp.float32), (2 * _K,))
    out = _time_to_interval(tv, flat)
    return jnp.reshape(out, ())


# speedup vs baseline: 1.0742x; 1.0031x over previous
import functools

import jax
import jax.numpy as jnp
from jax import lax
from jax.experimental import pallas as pl
from jax.experimental.pallas import tpu as pltpu
from jax.experimental.pallas import tpu_sc as plsc

_K = 128
_L = 16
_CHUNKS = _K // _L


def _sc_body(t_hbm, lo_hbm, hi_hbm, out_hbm, t_v, lo_v, hi_v, out_v,
             sem_t, sem_lo, sem_hi):
    cid = lax.axis_index("c")
    sid = lax.axis_index("s")

    @pl.when(jnp.logical_and(cid == 0, sid == 0))
    def _():
        cp_t = pltpu.make_async_copy(t_hbm, t_v, sem_t)
        cp_lo = pltpu.make_async_copy(lo_hbm, lo_v, sem_lo)
        cp_hi = pltpu.make_async_copy(hi_hbm, hi_v, sem_hi)
        cp_t.start()
        cp_lo.start()
        cp_hi.start()
        cp_t.wait()
        cp_lo.wait()
        cp_hi.wait()

        tv = t_v[...]
        best = jnp.full((_L,), _K, dtype=jnp.int32)
        for i in range(_CHUNKS):
            lov = lo_v[pl.ds(i * _L, _L)]
            hiv = hi_v[pl.ds(i * _L, _L)]
            cond = jnp.logical_and(tv >= lov, tv <= hiv)
            ffs = plsc.all_reduce_ffs(cond)
            cand = jnp.where(ffs >= _L, _K, ffs + (i * _L))
            best = jnp.minimum(best, cand)

        res = jnp.where(best >= _K, jnp.full((_L,), -1, jnp.int32), best)
        out_v[...] = res
        pltpu.sync_copy(out_v.at[pl.ds(0, 1)], out_hbm)


@jax.jit
def _time_to_interval(t, intervals):
    tv = jnp.broadcast_to(jnp.asarray(t, jnp.float32), (_L,))
    lo = jnp.asarray(intervals[:, 0], jnp.float32)
    hi = jnp.asarray(intervals[:, 1], jnp.float32)
    run = functools.partial(
        pl.kernel,
        out_type=jax.ShapeDtypeStruct((1,), jnp.int32),
        mesh=plsc.VectorSubcoreMesh(
            core_axis_name="c", subcore_axis_name="s", num_cores=1,
            num_subcores=1
        ),
        compiler_params=pltpu.CompilerParams(
            needs_layout_passes=False, skip_device_barrier=True
        ),
        scratch_types=[
            pltpu.VMEM((_L,), jnp.float32),
            pltpu.VMEM((_K,), jnp.float32),
            pltpu.VMEM((_K,), jnp.float32),
            pltpu.VMEM((_L,), jnp.int32),
            pltpu.SemaphoreType.DMA,
            pltpu.SemaphoreType.DMA,
            pltpu.SemaphoreType.DMA,
        ],
    )(_sc_body)
    return run(tv, lo, hi)


def kernel(t, intervals):
    out = _time_to_interval(t, intervals)
    return jnp.reshape(out, ())


# speedup vs baseline: 1.1390x; 1.0604x over previous
import functools

import jax
import jax.numpy as jnp
from jax import lax
from jax.experimental import pallas as pl
from jax.experimental.pallas import tpu as pltpu
from jax.experimental.pallas import tpu_sc as plsc

_K = 128


def _scs_body(t_hbm, iv_hbm, out_hbm, t_s, iv_s, out_s):
    @pl.when(lax.axis_index("c") == 0)
    def _():
        pltpu.sync_copy(t_hbm, t_s)
        pltpu.sync_copy(iv_hbm, iv_s)
        t = t_s[0]

        def cond(state):
            i, found = state
            return jnp.logical_and(i < _K, jnp.logical_not(found))

        def step(state):
            i, _ = state
            hit = jnp.logical_and(t >= iv_s[2 * i], t <= iv_s[2 * i + 1])
            return jnp.where(hit, i, i + 1), hit

        i, found = lax.while_loop(cond, step, (jnp.int32(0), jnp.bool_(False)))
        out_s[0] = jnp.where(found, i, jnp.int32(-1))
        pltpu.sync_copy(out_s, out_hbm)


@jax.jit
def _time_to_interval(tv, flat):
    run = functools.partial(
        pl.kernel,
        out_type=jax.ShapeDtypeStruct((1,), jnp.int32),
        mesh=plsc.ScalarSubcoreMesh(axis_name="c", num_cores=1),
        compiler_params=pltpu.CompilerParams(
            needs_layout_passes=False, skip_device_barrier=True
        ),
        scratch_types=[
            pltpu.SMEM((1,), jnp.float32),
            pltpu.SMEM((2 * _K,), jnp.float32),
            pltpu.SMEM((1,), jnp.int32),
        ],
    )(_scs_body)
    return run(tv, flat)


def kernel(t, intervals):
    tv = jnp.reshape(jnp.asarray(t, jnp.float32), (1,))
    flat = jnp.reshape(jnp.asarray(intervals, jnp.float32), (2 * _K,))
    out = _time_to_interval(tv, flat)
    return jnp.reshape(out, ())


# speedup vs baseline: 1.1694x; 1.0266x over previous
import functools

import jax
import jax.numpy as jnp
from jax import lax
from jax.experimental import pallas as pl
from jax.experimental.pallas import tpu as pltpu
from jax.experimental.pallas import tpu_sc as plsc

_K = 128


def _scs_body(t_hbm, iv_hbm, out_hbm, t_s, iv_s, out_s, sem_t, sem_iv):
    @pl.when(lax.axis_index("c") == 0)
    def _():
        cp_t = pltpu.make_async_copy(t_hbm, t_s, sem_t)
        cp_iv = pltpu.make_async_copy(iv_hbm, iv_s, sem_iv)
        cp_t.start()
        cp_iv.start()
        cp_t.wait()
        cp_iv.wait()
        t = t_s[0]

        def cond(state):
            i, found = state
            return jnp.logical_and(i < _K, jnp.logical_not(found))

        def step(state):
            i, _ = state
            hit = jnp.logical_and(t >= iv_s[2 * i], t <= iv_s[2 * i + 1])
            return jnp.where(hit, i, i + 1), hit

        i, found = lax.while_loop(cond, step, (jnp.int32(0), jnp.bool_(False)))
        out_s[0] = jnp.where(found, i, jnp.int32(-1))
        pltpu.sync_copy(out_s, out_hbm)


@jax.jit
def _time_to_interval(tv, flat):
    run = functools.partial(
        pl.kernel,
        out_type=jax.ShapeDtypeStruct((1,), jnp.int32),
        mesh=plsc.ScalarSubcoreMesh(axis_name="c", num_cores=1),
        compiler_params=pltpu.CompilerParams(
            needs_layout_passes=False, skip_device_barrier=True
        ),
        scratch_types=[
            pltpu.SMEM((1,), jnp.float32),
            pltpu.SMEM((2 * _K,), jnp.float32),
            pltpu.SMEM((1,), jnp.int32),
            pltpu.SemaphoreType.DMA,
            pltpu.SemaphoreType.DMA,
        ],
    )(_scs_body)
    return run(tv, flat)


def kernel(t, intervals):
    tv = jnp.reshape(jnp.asarray(t, jnp.float32), (1,))
    flat = jnp.reshape(jnp.asarray(intervals, jnp.float32), (2 * _K,))
    out = _time_to_interval(tv, flat)
    return jnp.reshape(out, ())
